# baseline (speedup 1.0000x reference)
import functools

import jax
import jax.numpy as jnp
from jax import lax
from jax.experimental import pallas as pl
from jax.experimental.pallas import tpu as pltpu, tpu_sc as plsc

VOCAB = 1000000
EMBED_DIM = 64
BATCH = 4096
HIST = 200
B = BATCH * HIST

_info = plsc.get_sparse_core_info()
NC, NS = _info.num_cores, _info.num_subcores
NW = NC * NS
B_PER_W = B // NW
B_BLK = BATCH // NW
HS = 2
SB = HS * B_BLK
N_STRIPES = HIST // HS
N_PAIRS = N_STRIPES // 2


def _make_gather():
    mesh = plsc.VectorSubcoreMesh(core_axis_name="c", subcore_axis_name="s")

    @functools.partial(
        pl.kernel,
        mesh=mesh,
        out_type=jax.ShapeDtypeStruct((HIST, EMBED_DIM, BATCH), jnp.float32),
        compiler_params=pltpu.CompilerParams(use_tc_tiling_on_sc=False,
                                             needs_layout_passes=False),
        scratch_types=(
            [pltpu.VMEM((B_PER_W,), jnp.int32)]
            + [pltpu.VMEM((SB,), jnp.int32) for _ in range(2)]
            + [pltpu.VMEM((SB, EMBED_DIM), jnp.float32) for _ in range(2)]
            + [pltpu.VMEM((HS, EMBED_DIM, B_BLK), jnp.float32) for _ in range(2)]
            + [pltpu.SemaphoreType.DMA for _ in range(4)]
        ),
    )
    def gather_kernel(idx_hbm, table_hbm, out_hbm,
                      idx_v, sidx0, sidx1, g0, g1, t0, t1,
                      gs0, gs1, ts0, ts1):
        sidx = (sidx0, sidx1)
        gbuf = (g0, g1)
        tbuf = (t0, t1)
        gsem = (gs0, gs1)
        tsem = (ts0, ts1)

        wid = lax.axis_index("s") * NC + lax.axis_index("c")
        base = wid * B_PER_W
        bb = wid * B_BLK
        pltpu.sync_copy(idx_hbm.at[pl.ds(pl.multiple_of(base, 8), B_PER_W)],
                        idx_v)

        iota = jnp.arange(16, dtype=jnp.int32)
        iota_h = iota * HIST

        def build_sidx(s, sref):
            for h_local in range(HS):
                for b0 in range(0, B_BLK, 16):
                    addr = iota_h + (b0 * HIST + s * HS + h_local)
                    v = plsc.load_gather(idx_v, [addr])
                    sref[pl.ds(h_local * B_BLK + b0, 16)] = v

        def gather_copy(par):
            return pltpu.make_async_copy(table_hbm.at[sidx[par]], gbuf[par],
                                         gsem[par])

        def out_copy(par, s, h_local):
            dst = out_hbm.at[s * HS + h_local, :, pl.ds(bb, B_BLK)]
            return pltpu.make_async_copy(tbuf[par].at[h_local], dst,
                                         tsem[par])

        rvecs = [iota + (h * B_BLK + b0)
                 for h in range(HS) for b0 in range(0, B_BLK, 16)]

        def transpose(par):
            g = gbuf[par]
            t = tbuf[par]

            def d_body(d, carry):
                cvec = jnp.zeros((16,), jnp.int32) + d
                for k, (h, b0) in enumerate(
                        (h, b0) for h in range(HS)
                        for b0 in range(0, B_BLK, 16)):
                    v = plsc.load_gather(g, [rvecs[k], cvec])
                    t[h, d, pl.ds(b0, 16)] = v
                return carry

            lax.fori_loop(0, EMBED_DIM, d_body, 0)

        for par in range(2):
            build_sidx(par, sidx[par])
            gather_copy(par).start()

        def pair_body(p, carry):
            for par in range(2):
                s = p * 2 + par
                gather_copy(par).wait()

                @pl.when(p >= 1)
                def _drain_out():
                    for h_local in range(HS):
                        out_copy(par, s, h_local).wait()

                transpose(par)
                for h_local in range(HS):
                    out_copy(par, s, h_local).start()

                @pl.when(p <= N_PAIRS - 2)
                def _prefetch():
                    build_sidx(s + 2, sidx[par])
                    gather_copy(par).start()

            return carry

        lax.fori_loop(0, N_PAIRS, pair_body, 0)

        for par in range(2):
            for h_local in range(HS):
                out_copy(par, N_STRIPES - 2 + par, h_local).wait()

    return gather_kernel


_gather = _make_gather()


def kernel(input_variable, table):
    idx = input_variable.reshape(-1).astype(jnp.int32)
    out_hdb = _gather(idx, table)
    return jnp.transpose(out_hdb, (2, 0, 1))


# speedup vs baseline: 1.6074x; 1.6074x over previous
import functools

import jax
import jax.numpy as jnp
from jax import lax
from jax.experimental import pallas as pl
from jax.experimental.pallas import tpu as pltpu, tpu_sc as plsc

VOCAB = 1000000
EMBED_DIM = 64
BATCH = 4096
HIST = 200
B = BATCH * HIST

_info = plsc.get_sparse_core_info()
NC, NS = _info.num_cores, _info.num_subcores
NW = NC * NS
B_PER_W = B // NW
B_BLK = BATCH // NW
HS = 2
SB = HS * B_BLK
TP = B_BLK + 1
N_STRIPES = HIST // HS
N_PAIRS = N_STRIPES // 2


def _make_gather():
    mesh = plsc.VectorSubcoreMesh(core_axis_name="c", subcore_axis_name="s")

    @functools.partial(
        pl.kernel,
        mesh=mesh,
        out_type=jax.ShapeDtypeStruct((HIST, EMBED_DIM, BATCH), jnp.float32),
        compiler_params=pltpu.CompilerParams(use_tc_tiling_on_sc=False,
                                             needs_layout_passes=False),
        scratch_types=(
            [pltpu.VMEM((B_PER_W,), jnp.int32)]
            + [pltpu.VMEM((SB,), jnp.int32) for _ in range(2)]
            + [pltpu.VMEM((SB, EMBED_DIM), jnp.float32) for _ in range(2)]
            + [pltpu.VMEM((HS, EMBED_DIM, TP), jnp.float32) for _ in range(2)]
            + [pltpu.SemaphoreType.DMA for _ in range(4)]
        ),
    )
    def gather_kernel(idx_hbm, table_hbm, out_hbm,
                      idx_v, sidx0, sidx1, g0, g1, t0, t1,
                      gs0, gs1, ts0, ts1):
        sidx = (sidx0, sidx1)
        gbuf = (g0, g1)
        tbuf = (t0, t1)
        gsem = (gs0, gs1)
        tsem = (ts0, ts1)

        wid = lax.axis_index("s") * NC + lax.axis_index("c")
        base = wid * B_PER_W
        bb = wid * B_BLK
        pltpu.sync_copy(idx_hbm.at[pl.ds(pl.multiple_of(base, 8), B_PER_W)],
                        idx_v)

        iota = jnp.arange(16, dtype=jnp.int32)
        iota_h = iota * HIST

        def build_sidx(s, sref):
            for h_local in range(HS):
                for b0 in range(0, B_BLK, 16):
                    addr = iota_h + (b0 * HIST + s * HS + h_local)
                    v = plsc.load_gather(idx_v, [addr])
                    sref[pl.ds(h_local * B_BLK + b0, 16)] = v

        def gather_copy(par):
            return pltpu.make_async_copy(table_hbm.at[sidx[par]], gbuf[par],
                                         gsem[par])

        def out_copy(par, s, h_local):
            dst = out_hbm.at[s * HS + h_local, :, pl.ds(bb, B_BLK)]
            src_blk = tbuf[par].at[h_local, :, pl.ds(0, B_BLK)]
            return pltpu.make_async_copy(src_blk, dst, tsem[par])

        dvecs = [iota + d0 for d0 in range(0, EMBED_DIM, 16)]

        def transpose(par):
            g = gbuf[par]
            t = tbuf[par]
            for h_local in range(HS):
                hvec = jnp.full((16,), h_local, jnp.int32)

                def b_body(b, carry, h_local=h_local, hvec=hvec):
                    bvec = jnp.zeros((16,), jnp.int32) + b
                    for k in range(EMBED_DIM // 16):
                        v = g[h_local * B_BLK + b, pl.ds(k * 16, 16)]
                        plsc.store_scatter(t, [hvec, dvecs[k], bvec], v)
                    return carry

                lax.fori_loop(0, B_BLK, b_body, 0)

        for par in range(2):
            build_sidx(par, sidx[par])
            gather_copy(par).start()

        def pair_body(p, carry):
            for par in range(2):
                s = p * 2 + par
                gather_copy(par).wait()

                @pl.when(p >= 1)
                def _drain_out():
                    for h_local in range(HS):
                        out_copy(par, s, h_local).wait()

                transpose(par)
                for h_local in range(HS):
                    out_copy(par, s, h_local).start()

                @pl.when(p <= N_PAIRS - 2)
                def _prefetch():
                    build_sidx(s + 2, sidx[par])
                    gather_copy(par).start()

            return carry

        lax.fori_loop(0, N_PAIRS, pair_body, 0)

        for par in range(2):
            for h_local in range(HS):
                out_copy(par, N_STRIPES - 2 + par, h_local).wait()

    return gather_kernel


_gather = _make_gather()


def kernel(input_variable, table):
    idx = input_variable.reshape(-1).astype(jnp.int32)
    out_hdb = _gather(idx, table)
    return jnp.transpose(out_hdb, (2, 0, 1))


# speedup vs baseline: 1.6323x; 1.0155x over previous
import functools

import jax
import jax.numpy as jnp
from jax import lax
from jax.experimental import pallas as pl
from jax.experimental.pallas import tpu as pltpu, tpu_sc as plsc

VOCAB = 1000000
EMBED_DIM = 64
BATCH = 4096
HIST = 200
B = BATCH * HIST

_info = plsc.get_sparse_core_info()
NC, NS = _info.num_cores, _info.num_subcores
NW = NC * NS
B_PER_W = B // NW
B_BLK = BATCH // NW
HS = 2
SB = HS * B_BLK
TP = B_BLK + 1
N_STRIPES = HIST // HS
NG = 4
N_QUADS = N_STRIPES // NG


def _make_gather():
    mesh = plsc.VectorSubcoreMesh(core_axis_name="c", subcore_axis_name="s")

    @functools.partial(
        pl.kernel,
        mesh=mesh,
        out_type=jax.ShapeDtypeStruct((HIST, EMBED_DIM, BATCH), jnp.float32),
        compiler_params=pltpu.CompilerParams(use_tc_tiling_on_sc=False,
                                             needs_layout_passes=False),
        scratch_types=(
            [pltpu.VMEM((B_PER_W,), jnp.int32)]
            + [pltpu.VMEM((SB,), jnp.int32) for _ in range(NG)]
            + [pltpu.VMEM((SB, EMBED_DIM), jnp.float32) for _ in range(NG)]
            + [pltpu.VMEM((HS * EMBED_DIM, TP), jnp.float32) for _ in range(2)]
            + [pltpu.SemaphoreType.DMA for _ in range(NG + 2)]
        ),
    )
    def gather_kernel(idx_hbm, table_hbm, out_hbm, idx_v, *rest):
        sidx = rest[:NG]
        gbuf = rest[NG:2 * NG]
        tbuf = rest[2 * NG:2 * NG + 2]
        gsem = rest[2 * NG + 2:3 * NG + 2]
        tsem = rest[3 * NG + 2:3 * NG + 4]

        wid = lax.axis_index("s") * NC + lax.axis_index("c")
        base = wid * B_PER_W
        bb = wid * B_BLK
        pltpu.sync_copy(idx_hbm.at[pl.ds(pl.multiple_of(base, 8), B_PER_W)],
                        idx_v)

        iota = jnp.arange(16, dtype=jnp.int32)
        iota_h = iota * HIST

        def build_sidx(s, sref):
            for h_local in range(HS):
                for b0 in range(0, B_BLK, 16):
                    addr = iota_h + (b0 * HIST + s * HS + h_local)
                    v = plsc.load_gather(idx_v, [addr])
                    sref[pl.ds(h_local * B_BLK + b0, 16)] = v

        def gather_copy(gi):
            return pltpu.make_async_copy(table_hbm.at[sidx[gi]], gbuf[gi],
                                         gsem[gi])

        def out_copy(ti, s, h_local):
            dst = out_hbm.at[s * HS + h_local, :, pl.ds(bb, B_BLK)]
            src_blk = tbuf[ti].at[pl.ds(h_local * EMBED_DIM, EMBED_DIM),
                                  pl.ds(0, B_BLK)]
            return pltpu.make_async_copy(src_blk, dst, tsem[ti])

        rvecs = [[iota + (h * EMBED_DIM + d0)
                  for d0 in range(0, EMBED_DIM, 16)] for h in range(HS)]

        def transpose(gi, ti):
            g = gbuf[gi]
            t = tbuf[ti]
            for h_local in range(HS):

                def b_body(b, carry, h_local=h_local):
                    bvec = jnp.zeros((16,), jnp.int32) + b
                    for k in range(EMBED_DIM // 16):
                        v = g[h_local * B_BLK + b, pl.ds(k * 16, 16)]
                        plsc.store_scatter(t, [rvecs[h_local][k], bvec], v)
                    return carry

                lax.fori_loop(0, B_BLK, b_body, 0, unroll=4)

        for s0 in range(NG - 1):
            build_sidx(s0, sidx[s0])
            gather_copy(s0).start()

        def quad_body(p, carry):
            for par in range(NG):
                s = p * NG + par
                ti = par % 2
                gather_copy(par).wait()

                nxt = (par + NG - 1) % NG
                ns = s + NG - 1

                @pl.when(ns < N_STRIPES)
                def _prefetch():
                    build_sidx(ns, sidx[nxt])
                    gather_copy(nxt).start()

                @pl.when(s >= 2)
                def _drain_out():
                    for h_local in range(HS):
                        out_copy(ti, s, h_local).wait()

                transpose(par, ti)
                for h_local in range(HS):
                    out_copy(ti, s, h_local).start()

            return carry

        lax.fori_loop(0, N_QUADS, quad_body, 0)

        for par in range(2):
            for h_local in range(HS):
                out_copy(par, N_STRIPES - 2 + par, h_local).wait()

    return gather_kernel


_gather = _make_gather()


def kernel(input_variable, table):
    idx = input_variable.reshape(-1).astype(jnp.int32)
    out_hdb = _gather(idx, table)
    return jnp.transpose(out_hdb, (2, 0, 1))


# speedup vs baseline: 1.6524x; 1.0124x over previous
import functools

import jax
import jax.numpy as jnp
from jax import lax
from jax.experimental import pallas as pl
from jax.experimental.pallas import tpu as pltpu, tpu_sc as plsc

VOCAB = 1000000
EMBED_DIM = 64
BATCH = 4096
HIST = 200
B = BATCH * HIST

_info = plsc.get_sparse_core_info()
NC, NS = _info.num_cores, _info.num_subcores
NW = NC * NS
B_PER_W = B // NW
CHUNK = 200
N_CHUNKS = B_PER_W // CHUNK
NBUF = 4
N_GROUPS = N_CHUNKS // NBUF
DEPTH = 2


def _make_gather():
    mesh = plsc.VectorSubcoreMesh(core_axis_name="c", subcore_axis_name="s")

    @functools.partial(
        pl.kernel,
        mesh=mesh,
        out_type=jax.ShapeDtypeStruct((BATCH, HIST, EMBED_DIM), jnp.float32),
        compiler_params=pltpu.CompilerParams(use_tc_tiling_on_sc=False),
        scratch_types=(
            [pltpu.VMEM((B_PER_W,), jnp.int32)]
            + [pltpu.VMEM((CHUNK, EMBED_DIM), jnp.float32) for _ in range(NBUF)]
            + [pltpu.SemaphoreType.DMA for _ in range(2 * NBUF)]
        ),
    )
    def gather_kernel(idx_hbm, table_hbm, out_hbm, idx_v, *bufs_and_sems):
        rows = bufs_and_sems[:NBUF]
        gsem = bufs_and_sems[NBUF : 2 * NBUF]
        ssem = bufs_and_sems[2 * NBUF : 3 * NBUF]

        wid = lax.axis_index("s") * NC + lax.axis_index("c")
        base = wid * B_PER_W
        pltpu.sync_copy(idx_hbm.at[pl.ds(pl.multiple_of(base, 8), B_PER_W)], idx_v)

        def start_gather(j, b):
            idx_sl = idx_v.at[pl.ds(j * CHUNK, CHUNK)]
            pltpu.make_async_copy(table_hbm.at[idx_sl], rows[b], gsem[b]).start()

        def wait_gather(j, b):
            idx_sl = idx_v.at[pl.ds(j * CHUNK, CHUNK)]
            pltpu.make_async_copy(table_hbm.at[idx_sl], rows[b], gsem[b]).wait()

        base_b = wid * (B_PER_W // HIST)

        def out_slice(j):
            return out_hbm.at[base_b + j]

        for b in range(DEPTH):
            start_gather(b, b)

        def group_body(g, carry):
            for b in range(NBUF):
                j = g * NBUF + b
                wait_gather(j, b)
                pltpu.make_async_copy(rows[b], out_slice(j), ssem[b]).start()
                j2 = j + DEPTH
                b2 = (b + DEPTH) % NBUF

                @pl.when(jnp.logical_and(j2 >= NBUF, j2 < N_CHUNKS))
                def _wait_store():
                    pltpu.make_async_copy(rows[b2], out_slice(j2), ssem[b2]).wait()

                @pl.when(j2 < N_CHUNKS)
                def _start_gather():
                    start_gather(j2, b2)

            return carry

        lax.fori_loop(0, N_GROUPS, group_body, 0)

        for b in range(NBUF):
            j = (N_GROUPS - 1) * NBUF + b
            pltpu.make_async_copy(rows[b], out_slice(j), ssem[b]).wait()

    return gather_kernel


_gather = _make_gather()


def kernel(input_variable, table):
    idx = input_variable.reshape(-1).astype(jnp.int32)
    return _gather(idx, table)
